# baseline (speedup 1.0000x reference)
import jax
import jax.numpy as jnp
from jax import lax
from jax.experimental import pallas as pl
from jax.experimental.pallas import tpu as pltpu

N = 2048
D = 256
NGL = 3
Bn, Rn, Ln = 8, 32, 64
E = Rn * Ln
ET = 128
NT = E // ET


def _lrelu(x):
    return jnp.where(x >= 0, x, 0.01 * x)


def _d(a, b):
    return jnp.dot(a, b, preferred_element_type=jnp.float32)


def _split(x):
    hi = x.astype(jnp.bfloat16).astype(jnp.float32)
    return hi, x - hi


def _dot_oh(m, x):
    x_hi, x_lo = _split(x)
    return _d(m, x_hi) + _d(m, x_lo)


def _dot3(a, b):
    a_hi, a_lo = _split(a)
    b_hi, b_lo = _split(b)
    return _d(a_hi, b_hi) + _d(a_hi, b_lo) + _d(a_lo, b_hi)


def _body(srcc_ref, dstr_ref, dstc_ref, wemb_ref, bemb_ref,
          gcnW_ref, gcnb_ref,
          ghW1_ref, ghb1_ref, ghW2_ref, ghb2_ref, ghW3_ref, ghb3_ref,
          rhW1_ref, rhb1_ref, rhW2_ref, rhb2_ref, rhW3_ref, rhb3_ref,
          gout_ref, rout_ref):
    f32 = jnp.float32
    src_col = srcc_ref[0]
    dst_row = dstr_ref[0]
    dst_col = dstc_ref[0]

    n_as_lane = lax.broadcasted_iota(jnp.int32, (ET, N), 1)
    n_as_subl = lax.broadcasted_iota(jnp.int32, (N, ET), 0)
    r_sel = lax.broadcasted_iota(jnp.int32, (Rn, 1), 0)

    h = wemb_ref[...].astype(jnp.bfloat16).astype(f32) + bemb_ref[...]

    ucnt = jnp.zeros((1, N), f32)
    rcnt = jnp.zeros((Rn, N), f32)

    for i in range(NGL):
        agg = jnp.zeros((N, D), f32)
        for t in range(NT):
            s_c = src_col[t * ET:(t + 1) * ET, :]
            d_r = dst_row[:, t * ET:(t + 1) * ET]
            Ms = (n_as_lane == s_c).astype(f32)
            Md_nt = (n_as_subl == d_r).astype(f32)
            g_t = _dot_oh(Ms, h)
            agg = agg + _dot_oh(Md_nt, g_t)
            if i == 0:
                d_c = dst_col[t * ET:(t + 1) * ET, :]
                Md_et = (n_as_lane == d_c).astype(f32)
                both = Ms + Md_et
                ucnt = ucnt + jnp.sum(both, axis=0, keepdims=True)
                c0 = jnp.sum(both[:Ln], axis=0, keepdims=True)
                c1 = jnp.sum(both[Ln:], axis=0, keepdims=True)
                rcnt = rcnt + (r_sel == (2 * t)).astype(f32) * c0 \
                            + (r_sel == (2 * t + 1)).astype(f32) * c1
        W = gcnW_ref[i]
        b = gcnb_ref[i:i + 1, :]
        h = _lrelu(_d(h + agg, W) + b)

    used = (ucnt > 0).astype(f32)
    gdesc = _dot_oh(used, h) / 1000.0
    rmask = (rcnt > 0).astype(f32)
    counts = jnp.sum(rmask, axis=1, keepdims=True)
    rdesc = _d(rmask, h) / counts

    gh = _lrelu(_d(gdesc, ghW1_ref[...]) + ghb1_ref[...])
    gh = _lrelu(_d(gh, ghW2_ref[...]) + ghb2_ref[...])
    gpred = _d(gh, ghW3_ref[...]) + ghb3_ref[...]

    tiled = jnp.broadcast_to(_lrelu(gdesc), (Rn, D))
    rin = jnp.concatenate([tiled, rdesc], axis=1)
    rh = _lrelu(_d(rin, rhW1_ref[...]) + rhb1_ref[...])
    rh = _lrelu(_d(rh, rhW2_ref[...]) + rhb2_ref[...])
    rpred = _d(rh, rhW3_ref[...]) + rhb3_ref[...]

    gout_ref[...] = gpred.reshape(1, 1, 1)
    rout_ref[...] = rpred.reshape(1, Rn, 1)


def kernel(env_rep, batch_route_idxs, Wemb, bemb, gcn_W, gcn_b,
           gh_W1, gh_b1, gh_W2, gh_b2, gh_W3, gh_b3,
           rh_W1, rh_b1, rh_W2, rh_b2, rh_W3, rh_b3):
    src = batch_route_idxs[:, :, 0, :].reshape(Bn, E)
    dst = batch_route_idxs[:, :, 1, :].reshape(Bn, E)
    src_col = src.reshape(Bn, E, 1)
    dst_row = dst.reshape(Bn, 1, E)
    dst_col = dst.reshape(Bn, E, 1)

    rep2 = lambda shape: pl.BlockSpec(shape, lambda b: (0, 0))
    rep3 = lambda shape: pl.BlockSpec(shape, lambda b: (0, 0, 0))

    gpred, rpred = pl.pallas_call(
        _body,
        grid=(Bn,),
        in_specs=[
            pl.BlockSpec((1, E, 1), lambda b: (b, 0, 0)),
            pl.BlockSpec((1, 1, E), lambda b: (b, 0, 0)),
            pl.BlockSpec((1, E, 1), lambda b: (b, 0, 0)),
            rep2((N, D)),
            rep2((1, D)),
            rep3((NGL, D, D)),
            rep2((NGL, D)),
            rep2((D, D)), rep2((1, D)),
            rep2((D, D)), rep2((1, D)),
            rep2((D, 1)), rep2((1, 1)),
            rep2((2 * D, D)), rep2((1, D)),
            rep2((D, D)), rep2((1, D)),
            rep2((D, 1)), rep2((1, 1)),
        ],
        out_specs=[
            pl.BlockSpec((1, 1, 1), lambda b: (b, 0, 0)),
            pl.BlockSpec((1, Rn, 1), lambda b: (b, 0, 0)),
        ],
        out_shape=[
            jax.ShapeDtypeStruct((Bn, 1, 1), jnp.float32),
            jax.ShapeDtypeStruct((Bn, Rn, 1), jnp.float32),
        ],
        compiler_params=pltpu.CompilerParams(
            dimension_semantics=("arbitrary",),
        ),
    )(src_col, dst_row, dst_col, Wemb, bemb.reshape(1, D),
      gcn_W, gcn_b,
      gh_W1, gh_b1.reshape(1, D), gh_W2, gh_b2.reshape(1, D),
      gh_W3, gh_b3.reshape(1, 1),
      rh_W1, rh_b1.reshape(1, D), rh_W2, rh_b2.reshape(1, D),
      rh_W3, rh_b3.reshape(1, 1))
    return gpred.reshape(Bn, 1), rpred
